# baseline (speedup 1.0000x reference)
import jax
import jax.numpy as jnp
from jax import lax
from jax.experimental import pallas as pl
from jax.experimental.pallas import tpu as pltpu
from jax.experimental.pallas import tpu_sc as plsc

N_FACTORS = 64
BATCH = 16384
NUM_WORKERS = 32
B_PER_W = BATCH // NUM_WORKERS
IDX_CHUNK = 128
N_CHUNKS = B_PER_W // IDX_CHUNK
LANES = 16
N_PASSES = 2
B_PER_PASS = B_PER_W // N_PASSES
GROUPS_PER_PASS = B_PER_PASS // LANES

CB = 8192
Q4 = 253952
MAXB = 122


def _pack_body(a_ref, b_ref, c_ref, d_ref, o_ref):
    parts = []
    for ref in (a_ref, b_ref, c_ref, d_ref):
        xq = ref[...].astype(jnp.bfloat16)
        pq = pltpu.bitcast(xq, jnp.int32)
        parts.append(jnp.swapaxes(pq, 0, 1))
    o_ref[...] = jnp.concatenate(parts, axis=1)


def _pack(tT):
    def mk_map(q):
        return lambda i: (0, jnp.minimum(i + 31 * q, MAXB))
    return pl.pallas_call(
        _pack_body,
        grid=(Q4 // CB,),
        in_specs=[pl.BlockSpec((64, CB), mk_map(q)) for q in range(4)],
        out_specs=pl.BlockSpec((CB, 128), lambda i: (i, 0)),
        out_shape=jax.ShapeDtypeStruct((Q4, 128), jnp.int32),
    )(tT, tT, tT, tT)


def _body(sess_hbm, aid_hbm, asz_hbm, stbl_hbm, atbl_hbm, out_hbm,
          sidx_o, aidx_o, sidx_p, aidx_p, asz_v, srows, arows, out_v,
          sem_in, sem_s, sem_a):
    wid = lax.axis_index("c") * 16 + lax.axis_index("s")

    c1 = pltpu.async_copy(sess_hbm.at[wid], sidx_o, sem_in)
    c2 = pltpu.async_copy(aid_hbm.at[wid], aidx_o, sem_in)
    c3 = pltpu.async_copy(asz_hbm.at[wid], asz_v, sem_in)
    c1.wait()
    c2.wait()
    c3.wait()

    def quarter(v):
        q = (jnp.where(v >= Q4, 1, 0) + jnp.where(v >= 2 * Q4, 1, 0)
             + jnp.where(v >= 3 * Q4, 1, 0)).astype(jnp.int32)
        return q

    def remap(i, _):
        c = i // 8
        l = (i % 8) * 16
        ov = sidx_o[c, pl.ds(l, 16)]
        sidx_p[c, pl.ds(l, 16)] = ov - quarter(ov) * Q4
        av = aidx_o[c, pl.ds(l, 16)]
        aidx_p[c, pl.ds(l, 16)] = av - quarter(av) * Q4
        return 0
    lax.fori_loop(0, N_CHUNKS * 8, remap, 0)

    lane = jnp.arange(LANES, dtype=jnp.int32)

    for p in range(N_PASSES):
        copies = []
        for j in range(2):
            c = p * 2 + j
            copies.append(pltpu.async_copy(
                stbl_hbm.at[sidx_p.at[c]],
                srows.at[pl.ds(j * IDX_CHUNK, IDX_CHUNK)], sem_s))
            copies.append(pltpu.async_copy(
                atbl_hbm.at[aidx_p.at[c]],
                arows.at[pl.ds(j * IDX_CHUNK, IDX_CHUNK)], sem_a))
        for c in copies:
            c.wait()

        def group_body(g, _):
            row = g * LANES + lane
            ch = p * 2 + g // 8
            l = (g % 8) * 16
            sv_o = sidx_o[ch, pl.ds(l, 16)]
            av_o = aidx_o[ch, pl.ds(l, 16)]
            so = quarter(sv_o) * 32
            ao = quarter(av_o) * 32

            def col_body(f, acc):
                sw = plsc.load_gather(srows, [row, so + f])
                aw = plsc.load_gather(arows, [row, ao + f])
                s0, s1 = plsc.unpack(plsc.bitcast(sw, jnp.bfloat16),
                                     format=plsc.PackFormat.INTERLEAVED)
                a0, a1 = plsc.unpack(plsc.bitcast(aw, jnp.bfloat16),
                                     format=plsc.PackFormat.INTERLEAVED)
                return acc + s0 * a0 + s1 * a1

            acc = lax.fori_loop(0, N_FACTORS // 2, col_body,
                                jnp.zeros((LANES,), jnp.float32))
            scale = asz_v[pl.ds(p * B_PER_PASS + g * LANES, LANES)]
            out_v[pl.ds(p * B_PER_PASS + g * LANES, LANES)] = acc * scale
            return 0

        lax.fori_loop(0, GROUPS_PER_PASS, group_body, 0)

    pltpu.sync_copy(out_v, out_hbm.at[wid])


def kernel(session, aid, aid_size, session_table, aid_table):
    mesh = plsc.VectorSubcoreMesh(core_axis_name="c", subcore_axis_name="s")
    k = pl.kernel(
        _body,
        out_type=jax.ShapeDtypeStruct((NUM_WORKERS, B_PER_W), jnp.float32),
        mesh=mesh,
        compiler_params=pltpu.CompilerParams(
            needs_layout_passes=False, use_tc_tiling_on_sc=False),
        scratch_types=[
            pltpu.VMEM((N_CHUNKS, IDX_CHUNK), jnp.int32),
            pltpu.VMEM((N_CHUNKS, IDX_CHUNK), jnp.int32),
            pltpu.VMEM((N_CHUNKS, IDX_CHUNK), jnp.int32),
            pltpu.VMEM((N_CHUNKS, IDX_CHUNK), jnp.int32),
            pltpu.VMEM((B_PER_W,), jnp.float32),
            pltpu.VMEM((B_PER_PASS, 128), jnp.int32),
            pltpu.VMEM((B_PER_PASS, 128), jnp.int32),
            pltpu.VMEM((B_PER_W,), jnp.float32),
            pltpu.SemaphoreType.DMA,
            pltpu.SemaphoreType.DMA,
            pltpu.SemaphoreType.DMA,
        ],
    )
    sess = session.astype(jnp.int32).reshape(NUM_WORKERS, N_CHUNKS, IDX_CHUNK)
    aidr = aid.astype(jnp.int32).reshape(NUM_WORKERS, N_CHUNKS, IDX_CHUNK)
    aszr = aid_size.reshape(NUM_WORKERS, B_PER_W)
    ps = _pack(jnp.swapaxes(session_table, 0, 1))
    pa = _pack(jnp.swapaxes(aid_table, 0, 1))
    out = k(sess, aidr, aszr, ps, pa)
    return out.reshape(BATCH)


# speedup vs baseline: 1.6444x; 1.6444x over previous
import jax
import jax.numpy as jnp
from jax import lax
from jax.experimental import pallas as pl
from jax.experimental.pallas import tpu as pltpu
from jax.experimental.pallas import tpu_sc as plsc

N_FACTORS = 64
BATCH = 16384
NUM_WORKERS = 32
B_PER_W = BATCH // NUM_WORKERS
IDX_CHUNK = 128
N_CHUNKS = B_PER_W // IDX_CHUNK
LANES = 16
N_PASSES = 2
B_PER_PASS = B_PER_W // N_PASSES
GROUPS_PER_PASS = B_PER_PASS // LANES

CB = 24576
HALF = 516096
Q = HALF // 2
N_BLOCKS = 123


def _pack_body(a_ref, b_ref, o_ref):
    xa = jnp.swapaxes(a_ref[...].astype(jnp.bfloat16), 0, 1)
    xb = jnp.swapaxes(b_ref[...].astype(jnp.bfloat16), 0, 1)
    pa = pltpu.bitcast(xa, jnp.int32)
    pb = pltpu.bitcast(xb, jnp.int32)
    o_ref[...] = jnp.concatenate([pa, pb], axis=1)


def _pack(tT):
    return pl.pallas_call(
        _pack_body,
        grid=(HALF // CB,),
        in_specs=[
            pl.BlockSpec((64, CB), lambda i: (0, i)),
            pl.BlockSpec((64, CB), lambda i: (0, jnp.minimum(i + 21, 40))),
        ],
        out_specs=pl.BlockSpec((CB // 2, 128), lambda i: (i, 0)),
        out_shape=jax.ShapeDtypeStruct((Q, 128), jnp.int32),
    )(tT, tT)


def _body(sess_hbm, aid_hbm, asz_hbm, stbl_hbm, atbl_hbm, out_hbm,
          sidx_o, aidx_o, sidx_p, aidx_p, asz_v, srows, arows, out_v,
          sem_in, sem_s, sem_a):
    wid = lax.axis_index("c") * 16 + lax.axis_index("s")

    c1 = pltpu.async_copy(sess_hbm.at[wid], sidx_o, sem_in)
    c2 = pltpu.async_copy(aid_hbm.at[wid], aidx_o, sem_in)
    c3 = pltpu.async_copy(asz_hbm.at[wid], asz_v, sem_in)
    c1.wait()
    c2.wait()
    c3.wait()

    def remap(i, _):
        c = i // 8
        l = (i % 8) * 16
        ov = sidx_o[c, pl.ds(l, 16)]
        sidx_p[c, pl.ds(l, 16)] = (ov - jnp.where(
            ov >= HALF, jnp.int32(HALF), jnp.int32(0))) >> 1
        av = aidx_o[c, pl.ds(l, 16)]
        aidx_p[c, pl.ds(l, 16)] = (av - jnp.where(
            av >= HALF, jnp.int32(HALF), jnp.int32(0))) >> 1
        return 0
    lax.fori_loop(0, N_CHUNKS * 8, remap, 0)

    lane = jnp.arange(LANES, dtype=jnp.int32)

    for p in range(N_PASSES):
        copies = []
        for j in range(2):
            c = p * 2 + j
            copies.append(pltpu.async_copy(
                stbl_hbm.at[sidx_p.at[c]],
                srows.at[pl.ds(j * IDX_CHUNK, IDX_CHUNK)], sem_s))
            copies.append(pltpu.async_copy(
                atbl_hbm.at[aidx_p.at[c]],
                arows.at[pl.ds(j * IDX_CHUNK, IDX_CHUNK)], sem_a))
        for c in copies:
            c.wait()

        def group_body(g, _):
            row = g * LANES + lane
            ch = p * 2 + g // 8
            l = (g % 8) * 16
            sv_o = sidx_o[ch, pl.ds(l, 16)]
            av_o = aidx_o[ch, pl.ds(l, 16)]
            so = jnp.where(sv_o >= HALF, jnp.int32(N_FACTORS), jnp.int32(0))
            ao = jnp.where(av_o >= HALF, jnp.int32(N_FACTORS), jnp.int32(0))
            sp = (sv_o & 1) == 1
            ap = (av_o & 1) == 1

            def col_body(f, acc):
                sw = plsc.load_gather(srows, [row, so + f])
                aw = plsc.load_gather(arows, [row, ao + f])
                se, sod = plsc.unpack(plsc.bitcast(sw, jnp.bfloat16),
                                      format=plsc.PackFormat.INTERLEAVED)
                ae, aod = plsc.unpack(plsc.bitcast(aw, jnp.bfloat16),
                                      format=plsc.PackFormat.INTERLEAVED)
                s = jnp.where(sp, sod, se)
                a = jnp.where(ap, aod, ae)
                return acc + s * a

            acc = lax.fori_loop(0, N_FACTORS, col_body,
                                jnp.zeros((LANES,), jnp.float32))
            scale = asz_v[pl.ds(p * B_PER_PASS + g * LANES, LANES)]
            out_v[pl.ds(p * B_PER_PASS + g * LANES, LANES)] = acc * scale
            return 0

        lax.fori_loop(0, GROUPS_PER_PASS, group_body, 0)

    pltpu.sync_copy(out_v, out_hbm.at[wid])


def kernel(session, aid, aid_size, session_table, aid_table):
    mesh = plsc.VectorSubcoreMesh(core_axis_name="c", subcore_axis_name="s")
    k = pl.kernel(
        _body,
        out_type=jax.ShapeDtypeStruct((NUM_WORKERS, B_PER_W), jnp.float32),
        mesh=mesh,
        compiler_params=pltpu.CompilerParams(
            needs_layout_passes=False, use_tc_tiling_on_sc=False),
        scratch_types=[
            pltpu.VMEM((N_CHUNKS, IDX_CHUNK), jnp.int32),
            pltpu.VMEM((N_CHUNKS, IDX_CHUNK), jnp.int32),
            pltpu.VMEM((N_CHUNKS, IDX_CHUNK), jnp.int32),
            pltpu.VMEM((N_CHUNKS, IDX_CHUNK), jnp.int32),
            pltpu.VMEM((B_PER_W,), jnp.float32),
            pltpu.VMEM((B_PER_PASS, 128), jnp.int32),
            pltpu.VMEM((B_PER_PASS, 128), jnp.int32),
            pltpu.VMEM((B_PER_W,), jnp.float32),
            pltpu.SemaphoreType.DMA,
            pltpu.SemaphoreType.DMA,
            pltpu.SemaphoreType.DMA,
        ],
    )
    sess = session.astype(jnp.int32).reshape(NUM_WORKERS, N_CHUNKS, IDX_CHUNK)
    aidr = aid.astype(jnp.int32).reshape(NUM_WORKERS, N_CHUNKS, IDX_CHUNK)
    aszr = aid_size.reshape(NUM_WORKERS, B_PER_W)
    ps = _pack(jnp.swapaxes(session_table, 0, 1))
    pa = _pack(jnp.swapaxes(aid_table, 0, 1))
    out = k(sess, aidr, aszr, ps, pa)
    return out.reshape(BATCH)


# speedup vs baseline: 1.6795x; 1.0214x over previous
import jax
import jax.numpy as jnp
from jax import lax
from jax.experimental import pallas as pl
from jax.experimental.pallas import tpu as pltpu
from jax.experimental.pallas import tpu_sc as plsc

N_FACTORS = 64
BATCH = 16384
NUM_WORKERS = 32
B_PER_W = BATCH // NUM_WORKERS
IDX_CHUNK = 128
N_CHUNKS = B_PER_W // IDX_CHUNK
LANES = 16
N_PASSES = 2
B_PER_PASS = B_PER_W // N_PASSES
GROUPS_PER_PASS = B_PER_PASS // LANES

CB = 24576
HALF = 516096
Q = HALF // 2
N_BLOCKS = 123


def _pack_body(a_ref, b_ref, o_ref):
    xa = jnp.swapaxes(a_ref[...].astype(jnp.bfloat16), 0, 1)
    xb = jnp.swapaxes(b_ref[...].astype(jnp.bfloat16), 0, 1)
    pa = pltpu.bitcast(xa, jnp.int32)
    pb = pltpu.bitcast(xb, jnp.int32)
    o_ref[...] = jnp.concatenate([pa, pb], axis=1)


def _pack(tT):
    return pl.pallas_call(
        _pack_body,
        grid=(HALF // CB,),
        in_specs=[
            pl.BlockSpec((64, CB), lambda i: (0, i)),
            pl.BlockSpec((64, CB), lambda i: (0, jnp.minimum(i + 21, 40))),
        ],
        out_specs=pl.BlockSpec((CB // 2, 128), lambda i: (i, 0)),
        out_shape=jax.ShapeDtypeStruct((Q, 128), jnp.int32),
    )(tT, tT)


def _body(sess_hbm, aid_hbm, asz_hbm, stbl_hbm, atbl_hbm, out_hbm,
          sidx_o, aidx_o, sidx_p, aidx_p, asz_v,
          srows0, srows1, arows0, arows1, out_v,
          sem_in, sem_s0, sem_s1, sem_a0, sem_a1):
    wid = lax.axis_index("c") * 16 + lax.axis_index("s")

    c1 = pltpu.async_copy(sess_hbm.at[wid], sidx_o, sem_in)
    c2 = pltpu.async_copy(aid_hbm.at[wid], aidx_o, sem_in)
    c3 = pltpu.async_copy(asz_hbm.at[wid], asz_v, sem_in)
    c1.wait()
    c2.wait()
    c3.wait()

    def remap(i, _):
        c = i // 8
        l = (i % 8) * 16
        ov = sidx_o[c, pl.ds(l, 16)]
        sidx_p[c, pl.ds(l, 16)] = (ov - jnp.where(
            ov >= HALF, jnp.int32(HALF), jnp.int32(0))) >> 1
        av = aidx_o[c, pl.ds(l, 16)]
        aidx_p[c, pl.ds(l, 16)] = (av - jnp.where(
            av >= HALF, jnp.int32(HALF), jnp.int32(0))) >> 1
        return 0
    lax.fori_loop(0, N_CHUNKS * 8, remap, 0)

    lane = jnp.arange(LANES, dtype=jnp.int32)
    sbufs = [srows0, srows1]
    abufs = [arows0, arows1]
    ssems = [sem_s0, sem_s1]
    asems = [sem_a0, sem_a1]

    def fire(c):
        return (pltpu.async_copy(stbl_hbm.at[sidx_p.at[c]],
                                 sbufs[c % 2], ssems[c % 2]),
                pltpu.async_copy(atbl_hbm.at[aidx_p.at[c]],
                                 abufs[c % 2], asems[c % 2]))

    handles = {0: fire(0)}
    for c in range(N_CHUNKS):
        if c + 1 < N_CHUNKS:
            handles[c + 1] = fire(c + 1)
        for h in handles[c]:
            h.wait()
        srows = sbufs[c % 2]
        arows = abufs[c % 2]

        def group_body(g, _):
            row = g * LANES + lane
            sv_o = sidx_o[c, pl.ds(g * LANES, 16)]
            av_o = aidx_o[c, pl.ds(g * LANES, 16)]
            so = jnp.where(sv_o >= HALF, jnp.int32(N_FACTORS), jnp.int32(0))
            ao = jnp.where(av_o >= HALF, jnp.int32(N_FACTORS), jnp.int32(0))
            sp = (sv_o & 1) == 1
            ap = (av_o & 1) == 1

            def one(f, acc):
                sw = plsc.load_gather(srows, [row, so + f])
                aw = plsc.load_gather(arows, [row, ao + f])
                se, sod = plsc.unpack(plsc.bitcast(sw, jnp.bfloat16),
                                      format=plsc.PackFormat.INTERLEAVED)
                ae, aod = plsc.unpack(plsc.bitcast(aw, jnp.bfloat16),
                                      format=plsc.PackFormat.INTERLEAVED)
                s = jnp.where(sp, sod, se)
                a = jnp.where(ap, aod, ae)
                return acc + s * a

            def col_body(f4, acc):
                f = f4 * 4
                acc = one(f, acc)
                acc = one(f + 1, acc)
                acc = one(f + 2, acc)
                acc = one(f + 3, acc)
                return acc

            acc = lax.fori_loop(0, N_FACTORS // 4, col_body,
                                jnp.zeros((LANES,), jnp.float32))
            scale = asz_v[pl.ds(c * IDX_CHUNK + g * LANES, LANES)]
            out_v[pl.ds(c * IDX_CHUNK + g * LANES, LANES)] = acc * scale
            return 0

        lax.fori_loop(0, IDX_CHUNK // LANES, group_body, 0)

    pltpu.sync_copy(out_v, out_hbm.at[wid])


def kernel(session, aid, aid_size, session_table, aid_table):
    mesh = plsc.VectorSubcoreMesh(core_axis_name="c", subcore_axis_name="s")
    k = pl.kernel(
        _body,
        out_type=jax.ShapeDtypeStruct((NUM_WORKERS, B_PER_W), jnp.float32),
        mesh=mesh,
        compiler_params=pltpu.CompilerParams(
            needs_layout_passes=False, use_tc_tiling_on_sc=False),
        scratch_types=[
            pltpu.VMEM((N_CHUNKS, IDX_CHUNK), jnp.int32),
            pltpu.VMEM((N_CHUNKS, IDX_CHUNK), jnp.int32),
            pltpu.VMEM((N_CHUNKS, IDX_CHUNK), jnp.int32),
            pltpu.VMEM((N_CHUNKS, IDX_CHUNK), jnp.int32),
            pltpu.VMEM((B_PER_W,), jnp.float32),
            pltpu.VMEM((IDX_CHUNK, 128), jnp.int32),
            pltpu.VMEM((IDX_CHUNK, 128), jnp.int32),
            pltpu.VMEM((IDX_CHUNK, 128), jnp.int32),
            pltpu.VMEM((IDX_CHUNK, 128), jnp.int32),
            pltpu.VMEM((B_PER_W,), jnp.float32),
            pltpu.SemaphoreType.DMA,
            pltpu.SemaphoreType.DMA,
            pltpu.SemaphoreType.DMA,
            pltpu.SemaphoreType.DMA,
            pltpu.SemaphoreType.DMA,
        ],
    )
    sess = session.astype(jnp.int32).reshape(NUM_WORKERS, N_CHUNKS, IDX_CHUNK)
    aidr = aid.astype(jnp.int32).reshape(NUM_WORKERS, N_CHUNKS, IDX_CHUNK)
    aszr = aid_size.reshape(NUM_WORKERS, B_PER_W)
    ps = _pack(jnp.swapaxes(session_table, 0, 1))
    pa = _pack(jnp.swapaxes(aid_table, 0, 1))
    out = k(sess, aidr, aszr, ps, pa)
    return out.reshape(BATCH)


# speedup vs baseline: 1.7530x; 1.0437x over previous
import jax
import jax.numpy as jnp
from jax import lax
from jax.experimental import pallas as pl
from jax.experimental.pallas import tpu as pltpu
from jax.experimental.pallas import tpu_sc as plsc

N_FACTORS = 64
BATCH = 16384
NUM_WORKERS = 32
B_PER_W = BATCH // NUM_WORKERS
IDX_CHUNK = 128
N_CHUNKS = B_PER_W // IDX_CHUNK
LANES = 16
N_PASSES = 2
B_PER_PASS = B_PER_W // N_PASSES
GROUPS_PER_PASS = B_PER_PASS // LANES

CB = 12288
HALF = 516096
Q = HALF // 2
N_BLOCKS = 123


def _pack_half(a_ref, b_ref):
    xa = jnp.swapaxes(a_ref[...].astype(jnp.bfloat16), 0, 1)
    xb = jnp.swapaxes(b_ref[...].astype(jnp.bfloat16), 0, 1)
    pa = pltpu.bitcast(xa, jnp.int32)
    pb = pltpu.bitcast(xb, jnp.int32)
    return jnp.concatenate([pa, pb], axis=1)


def _pack_body(sa_ref, sb_ref, aa_ref, ab_ref, os_ref, oa_ref):
    os_ref[...] = _pack_half(sa_ref, sb_ref)
    oa_ref[...] = _pack_half(aa_ref, ab_ref)


def _pack2(tTs, tTa):
    lo = lambda i: (0, i)
    hi = lambda i: (0, jnp.minimum(i + 42, 81))
    return pl.pallas_call(
        _pack_body,
        grid=(HALF // CB,),
        in_specs=[
            pl.BlockSpec((64, CB), lo),
            pl.BlockSpec((64, CB), hi),
            pl.BlockSpec((64, CB), lo),
            pl.BlockSpec((64, CB), hi),
        ],
        out_specs=[pl.BlockSpec((CB // 2, 128), lambda i: (i, 0))] * 2,
        out_shape=[jax.ShapeDtypeStruct((Q, 128), jnp.int32)] * 2,
    )(tTs, tTs, tTa, tTa)


def _body(sess_hbm, aid_hbm, asz_hbm, stbl_hbm, atbl_hbm, out_hbm,
          sidx_o, aidx_o, sidx_p, aidx_p, asz_v,
          srows0, srows1, arows0, arows1, out_v,
          sem_in, sem_s0, sem_s1, sem_a0, sem_a1):
    wid = lax.axis_index("c") * 16 + lax.axis_index("s")

    c1 = pltpu.async_copy(sess_hbm.at[wid], sidx_o, sem_in)
    c2 = pltpu.async_copy(aid_hbm.at[wid], aidx_o, sem_in)
    c3 = pltpu.async_copy(asz_hbm.at[wid], asz_v, sem_in)
    c1.wait()
    c2.wait()
    c3.wait()

    def remap(i, _):
        c = i // 8
        l = (i % 8) * 16
        ov = sidx_o[c, pl.ds(l, 16)]
        sidx_p[c, pl.ds(l, 16)] = (ov - jnp.where(
            ov >= HALF, jnp.int32(HALF), jnp.int32(0))) >> 1
        av = aidx_o[c, pl.ds(l, 16)]
        aidx_p[c, pl.ds(l, 16)] = (av - jnp.where(
            av >= HALF, jnp.int32(HALF), jnp.int32(0))) >> 1
        return 0
    lax.fori_loop(0, N_CHUNKS * 8, remap, 0)

    lane = jnp.arange(LANES, dtype=jnp.int32)
    sbufs = [srows0, srows1]
    abufs = [arows0, arows1]
    ssems = [sem_s0, sem_s1]
    asems = [sem_a0, sem_a1]

    def fire(c):
        return (pltpu.async_copy(stbl_hbm.at[sidx_p.at[c]],
                                 sbufs[c % 2], ssems[c % 2]),
                pltpu.async_copy(atbl_hbm.at[aidx_p.at[c]],
                                 abufs[c % 2], asems[c % 2]))

    handles = {0: fire(0)}
    for c in range(N_CHUNKS):
        if c + 1 < N_CHUNKS:
            handles[c + 1] = fire(c + 1)
        for h in handles[c]:
            h.wait()
        srows = sbufs[c % 2]
        arows = abufs[c % 2]

        def group_body(g, _):
            row = g * LANES + lane
            sv_o = sidx_o[c, pl.ds(g * LANES, 16)]
            av_o = aidx_o[c, pl.ds(g * LANES, 16)]
            so = jnp.where(sv_o >= HALF, jnp.int32(N_FACTORS), jnp.int32(0))
            ao = jnp.where(av_o >= HALF, jnp.int32(N_FACTORS), jnp.int32(0))
            sp = (sv_o & 1) == 1
            ap = (av_o & 1) == 1

            def one(f, acc):
                sw = plsc.load_gather(srows, [row, so + f])
                aw = plsc.load_gather(arows, [row, ao + f])
                se, sod = plsc.unpack(plsc.bitcast(sw, jnp.bfloat16),
                                      format=plsc.PackFormat.INTERLEAVED)
                ae, aod = plsc.unpack(plsc.bitcast(aw, jnp.bfloat16),
                                      format=plsc.PackFormat.INTERLEAVED)
                s = jnp.where(sp, sod, se)
                a = jnp.where(ap, aod, ae)
                return acc + s * a

            def col_body(f4, acc):
                f = f4 * 4
                acc = one(f, acc)
                acc = one(f + 1, acc)
                acc = one(f + 2, acc)
                acc = one(f + 3, acc)
                return acc

            acc = lax.fori_loop(0, N_FACTORS // 4, col_body,
                                jnp.zeros((LANES,), jnp.float32))
            scale = asz_v[pl.ds(c * IDX_CHUNK + g * LANES, LANES)]
            out_v[pl.ds(c * IDX_CHUNK + g * LANES, LANES)] = acc * scale
            return 0

        lax.fori_loop(0, IDX_CHUNK // LANES, group_body, 0)

    pltpu.sync_copy(out_v, out_hbm.at[wid])


def kernel(session, aid, aid_size, session_table, aid_table):
    mesh = plsc.VectorSubcoreMesh(core_axis_name="c", subcore_axis_name="s")
    k = pl.kernel(
        _body,
        out_type=jax.ShapeDtypeStruct((NUM_WORKERS, B_PER_W), jnp.float32),
        mesh=mesh,
        compiler_params=pltpu.CompilerParams(
            needs_layout_passes=False, use_tc_tiling_on_sc=False),
        scratch_types=[
            pltpu.VMEM((N_CHUNKS, IDX_CHUNK), jnp.int32),
            pltpu.VMEM((N_CHUNKS, IDX_CHUNK), jnp.int32),
            pltpu.VMEM((N_CHUNKS, IDX_CHUNK), jnp.int32),
            pltpu.VMEM((N_CHUNKS, IDX_CHUNK), jnp.int32),
            pltpu.VMEM((B_PER_W,), jnp.float32),
            pltpu.VMEM((IDX_CHUNK, 128), jnp.int32),
            pltpu.VMEM((IDX_CHUNK, 128), jnp.int32),
            pltpu.VMEM((IDX_CHUNK, 128), jnp.int32),
            pltpu.VMEM((IDX_CHUNK, 128), jnp.int32),
            pltpu.VMEM((B_PER_W,), jnp.float32),
            pltpu.SemaphoreType.DMA,
            pltpu.SemaphoreType.DMA,
            pltpu.SemaphoreType.DMA,
            pltpu.SemaphoreType.DMA,
            pltpu.SemaphoreType.DMA,
        ],
    )
    sess = session.astype(jnp.int32).reshape(NUM_WORKERS, N_CHUNKS, IDX_CHUNK)
    aidr = aid.astype(jnp.int32).reshape(NUM_WORKERS, N_CHUNKS, IDX_CHUNK)
    aszr = aid_size.reshape(NUM_WORKERS, B_PER_W)
    ps, pa = _pack2(jnp.swapaxes(session_table, 0, 1),
                    jnp.swapaxes(aid_table, 0, 1))
    out = k(sess, aidr, aszr, ps, pa)
    return out.reshape(BATCH)


# speedup vs baseline: 1.7540x; 1.0006x over previous
import jax
import jax.numpy as jnp
from jax import lax
from jax.experimental import pallas as pl
from jax.experimental.pallas import tpu as pltpu
from jax.experimental.pallas import tpu_sc as plsc

N_FACTORS = 64
BATCH = 16384
NUM_WORKERS = 32
B_PER_W = BATCH // NUM_WORKERS
IDX_CHUNK = 128
N_CHUNKS = B_PER_W // IDX_CHUNK
LANES = 16
N_PASSES = 2
B_PER_PASS = B_PER_W // N_PASSES
GROUPS_PER_PASS = B_PER_PASS // LANES

CB = 12288
HALF = 516096
Q = HALF // 2


def _pack_half(a_ref, b_ref):
    xa = jnp.swapaxes(a_ref[...].astype(jnp.bfloat16), 0, 1)
    xb = jnp.swapaxes(b_ref[...].astype(jnp.bfloat16), 0, 1)
    pa = pltpu.bitcast(xa, jnp.int32)
    pb = pltpu.bitcast(xb, jnp.int32)
    return jnp.concatenate([pa, pb], axis=1)


def _pack_body(sa_ref, sb_ref, aa_ref, ab_ref, os_ref, oa_ref):
    os_ref[...] = _pack_half(sa_ref, sb_ref)
    oa_ref[...] = _pack_half(aa_ref, ab_ref)


def _pack2(tTs, tTa):
    lo = lambda i: (0, i)
    hi = lambda i: (0, jnp.minimum(i + 42, 81))
    return pl.pallas_call(
        _pack_body,
        grid=(HALF // CB,),
        in_specs=[
            pl.BlockSpec((64, CB), lo),
            pl.BlockSpec((64, CB), hi),
            pl.BlockSpec((64, CB), lo),
            pl.BlockSpec((64, CB), hi),
        ],
        out_specs=[pl.BlockSpec((CB // 2, 128), lambda i: (i, 0))] * 2,
        out_shape=[jax.ShapeDtypeStruct((Q, 128), jnp.int32)] * 2,
    )(tTs, tTs, tTa, tTa)


def _body(sess_hbm, aid_hbm, asz_hbm, stbl_hbm, atbl_hbm, out_hbm,
          sidx_o, aidx_o, sidx_p, aidx_p, asz_v,
          srows0, srows1, arows0, arows1, out_v,
          sem_in, sem_s0, sem_s1, sem_a0, sem_a1):
    wid = lax.axis_index("c") * 16 + lax.axis_index("s")

    c1 = pltpu.async_copy(sess_hbm.at[wid], sidx_o, sem_in)
    c2 = pltpu.async_copy(aid_hbm.at[wid], aidx_o, sem_in)
    c3 = pltpu.async_copy(asz_hbm.at[wid], asz_v, sem_in)
    c1.wait()
    c2.wait()
    c3.wait()

    def remap(i, _):
        c = i // 8
        l = (i % 8) * 16
        ov = sidx_o[c, pl.ds(l, 16)]
        sidx_p[c, pl.ds(l, 16)] = (ov - jnp.where(
            ov >= HALF, jnp.int32(HALF), jnp.int32(0))) >> 1
        av = aidx_o[c, pl.ds(l, 16)]
        aidx_p[c, pl.ds(l, 16)] = (av - jnp.where(
            av >= HALF, jnp.int32(HALF), jnp.int32(0))) >> 1
        return 0
    lax.fori_loop(0, N_CHUNKS * 8, remap, 0)

    lane = jnp.arange(LANES, dtype=jnp.int32)
    sbufs = [srows0, srows1]
    abufs = [arows0, arows1]
    ssems = [sem_s0, sem_s1]
    asems = [sem_a0, sem_a1]

    def fire(c):
        return (pltpu.async_copy(stbl_hbm.at[sidx_p.at[c]],
                                 sbufs[c % 2], ssems[c % 2]),
                pltpu.async_copy(atbl_hbm.at[aidx_p.at[c]],
                                 abufs[c % 2], asems[c % 2]))

    handles = {0: fire(0)}
    for c in range(N_CHUNKS):
        if c + 1 < N_CHUNKS:
            handles[c + 1] = fire(c + 1)
        for h in handles[c]:
            h.wait()
        srows = sbufs[c % 2]
        arows = abufs[c % 2]

        def group_body(g, _):
            row = g * LANES + lane
            sv_o = sidx_o[c, pl.ds(g * LANES, 16)]
            av_o = aidx_o[c, pl.ds(g * LANES, 16)]
            so = jnp.where(sv_o >= HALF, jnp.int32(N_FACTORS), jnp.int32(0))
            ao = jnp.where(av_o >= HALF, jnp.int32(N_FACTORS), jnp.int32(0))
            sp = (sv_o & 1) == 1
            ap = (av_o & 1) == 1

            def one(f, acc):
                sw = plsc.load_gather(srows, [row, so + f])
                aw = plsc.load_gather(arows, [row, ao + f])
                se, sod = plsc.unpack(plsc.bitcast(sw, jnp.bfloat16),
                                      format=plsc.PackFormat.INTERLEAVED)
                ae, aod = plsc.unpack(plsc.bitcast(aw, jnp.bfloat16),
                                      format=plsc.PackFormat.INTERLEAVED)
                s = jnp.where(sp, sod, se)
                a = jnp.where(ap, aod, ae)
                return acc + s * a

            def col_body(f4, acc):
                f = f4 * 4
                acc = one(f, acc)
                acc = one(f + 1, acc)
                acc = one(f + 2, acc)
                acc = one(f + 3, acc)
                return acc

            acc = lax.fori_loop(0, N_FACTORS // 4, col_body,
                                jnp.zeros((LANES,), jnp.float32))
            scale = asz_v[pl.ds(c * IDX_CHUNK + g * LANES, LANES)]
            out_v[pl.ds(c * IDX_CHUNK + g * LANES, LANES)] = acc * scale
            return 0

        lax.fori_loop(0, IDX_CHUNK // LANES, group_body, 0)

    pltpu.sync_copy(out_v, out_hbm.at[wid])


def kernel(session, aid, aid_size, session_table, aid_table):
    mesh = plsc.VectorSubcoreMesh(core_axis_name="c", subcore_axis_name="s")
    k = pl.kernel(
        _body,
        out_type=jax.ShapeDtypeStruct((NUM_WORKERS, B_PER_W), jnp.float32),
        mesh=mesh,
        compiler_params=pltpu.CompilerParams(
            needs_layout_passes=False, use_tc_tiling_on_sc=False),
        scratch_types=[
            pltpu.VMEM((N_CHUNKS, IDX_CHUNK), jnp.int32),
            pltpu.VMEM((N_CHUNKS, IDX_CHUNK), jnp.int32),
            pltpu.VMEM((N_CHUNKS, IDX_CHUNK), jnp.int32),
            pltpu.VMEM((N_CHUNKS, IDX_CHUNK), jnp.int32),
            pltpu.VMEM((B_PER_W,), jnp.float32),
            pltpu.VMEM((IDX_CHUNK, 128), jnp.int32),
            pltpu.VMEM((IDX_CHUNK, 128), jnp.int32),
            pltpu.VMEM((IDX_CHUNK, 128), jnp.int32),
            pltpu.VMEM((IDX_CHUNK, 128), jnp.int32),
            pltpu.VMEM((B_PER_W,), jnp.float32),
            pltpu.SemaphoreType.DMA,
            pltpu.SemaphoreType.DMA,
            pltpu.SemaphoreType.DMA,
            pltpu.SemaphoreType.DMA,
            pltpu.SemaphoreType.DMA,
        ],
    )
    sess = session.astype(jnp.int32).reshape(NUM_WORKERS, N_CHUNKS, IDX_CHUNK)
    aidr = aid.astype(jnp.int32).reshape(NUM_WORKERS, N_CHUNKS, IDX_CHUNK)
    aszr = aid_size.reshape(NUM_WORKERS, B_PER_W)
    ps, pa = _pack2(jnp.swapaxes(session_table, 0, 1),
                    jnp.swapaxes(aid_table, 0, 1))
    out = k(sess, aidr, aszr, ps, pa)
    return out.reshape(BATCH)
